# Initial kernel scaffold; baseline (speedup 1.0000x reference)
import jax
import jax.numpy as jnp
from jax.experimental import pallas as pl


def kernel(x, fc, bias, offsets):
    raise NotImplementedError("write your pallas kernel here")


# speedup vs baseline: 1.1970x; 1.1970x over previous
import functools

import jax
import jax.numpy as jnp
from jax import lax
from jax.experimental import pallas as pl
from jax.experimental.pallas import tpu as pltpu
from jax.experimental.pallas import tpu_sc as plsc

B = 16384
F = 26
NC, NS, L = 2, 16, 16
NW = NC * NS
BW = B // NW
NIDX = F * BW
CHUNK = 128
NCH = BW // CHUNK


def _sc_body(xt_hbm, fc_hbm, offb_hbm, biasb_hbm, out_hbm,
             xv, rows, outv, offv, biasv, sem):
    wid = lax.axis_index("s") * NC + lax.axis_index("c")
    base = wid * BW

    pltpu.sync_copy(xt_hbm.at[wid], xv)
    pltpu.sync_copy(offb_hbm, offv)
    pltpu.sync_copy(biasb_hbm, biasv)

    @pl.loop(0, BW // L)
    def _idx(c):
        for f in range(F):
            xv[f, pl.ds(c * L, L)] = xv[f, pl.ds(c * L, L)] + offv[f, :]

    @pl.loop(0, F)
    def _gather(f):
        for c4 in range(NCH):
            pltpu.make_async_copy(
                fc_hbm.at[xv.at[f, pl.ds(c4 * CHUNK, CHUNK)]],
                rows.at[pl.ds(f * BW + c4 * CHUNK, CHUNK)],
                sem,
            ).start()
    pltpu.make_async_copy(fc_hbm.at[pl.ds(0, NIDX)], rows, sem).wait()

    bias_vec = biasv[:]
    @pl.loop(0, BW // L)
    def _reduce(c):
        acc = bias_vec
        for f in range(F):
            acc = acc + rows[pl.ds(f * BW + c * L, L)]
        outv[pl.ds(c * L, L)] = acc

    pltpu.sync_copy(outv, out_hbm.at[pl.ds(base, BW)])


@jax.jit
def _features_linear(xt, fc1, offb, biasb):
    mesh = plsc.VectorSubcoreMesh(core_axis_name="c", subcore_axis_name="s")
    return pl.kernel(
        _sc_body,
        out_type=jax.ShapeDtypeStruct((B,), jnp.float32),
        mesh=mesh,
        scratch_types=[
            pltpu.VMEM((F, BW), jnp.int32),
            pltpu.VMEM((NIDX,), jnp.float32),
            pltpu.VMEM((BW,), jnp.float32),
            pltpu.VMEM((F, L), jnp.int32),
            pltpu.VMEM((L,), jnp.float32),
            pltpu.SemaphoreType.DMA,
        ],
    )(xt, fc1, offb, biasb)


def kernel(x, fc, bias, offsets):
    xt = x.T.reshape(F, NW, BW).transpose(1, 0, 2)
    fc1 = fc.reshape(-1)
    offb = jnp.broadcast_to(offsets[:, None], (F, L))
    biasb = jnp.broadcast_to(bias, (L,))
    out = _features_linear(xt, fc1, offb, biasb)
    return out.reshape(B, 1)


# speedup vs baseline: 4.0116x; 3.3514x over previous
import functools

import jax
import jax.numpy as jnp
from jax import lax
from jax.experimental import pallas as pl
from jax.experimental.pallas import tpu as pltpu
from jax.experimental.pallas import tpu_sc as plsc

B = 16384
F = 26
NC, NS, L = 2, 16, 16
NW = NC * NS
BW = B // NW
NIDX = F * BW
CHUNK = 128
NCH = BW // CHUNK


def _sc_body(xt_hbm, fc_hbm, offb_hbm, biasb_hbm, out_hbm,
             xv, rows, outv, offv, biasv, sem):
    wid = lax.axis_index("s") * NC + lax.axis_index("c")
    base = wid * BW

    pltpu.sync_copy(xt_hbm.at[wid], xv)
    pltpu.sync_copy(offb_hbm, offv)
    pltpu.sync_copy(biasb_hbm, biasv)

    @pl.loop(0, BW // L)
    def _idx(c):
        for f in range(F):
            xv[f, pl.ds(c * L, L)] = xv[f, pl.ds(c * L, L)] + offv[f, :]

    @pl.loop(0, F)
    def _gather(f):
        for c4 in range(NCH):
            pltpu.make_async_copy(
                fc_hbm.at[xv.at[pl.ds(f, 1), pl.ds(c4 * CHUNK, CHUNK)]],
                rows.at[pl.ds(f, 1), pl.ds(c4 * CHUNK, CHUNK)],
                sem,
            ).start()

    @pl.loop(0, F)
    def _drain(f):
        for c4 in range(NCH):
            pltpu.make_async_copy(
                fc_hbm.at[xv.at[pl.ds(f, 1), pl.ds(c4 * CHUNK, CHUNK)]],
                rows.at[pl.ds(f, 1), pl.ds(c4 * CHUNK, CHUNK)],
                sem,
            ).wait()

    bias_vec = biasv[:]
    @pl.loop(0, BW // L)
    def _reduce(c):
        acc = bias_vec
        for f in range(F):
            acc = acc + rows[f, pl.ds(c * L, L)]
        outv[pl.ds(c * L, L)] = acc

    pltpu.sync_copy(outv, out_hbm.at[pl.ds(base, BW)])


@jax.jit
def _features_linear(xt, fcr, offb, biasb):
    mesh = plsc.VectorSubcoreMesh(core_axis_name="c", subcore_axis_name="s")
    return pl.kernel(
        _sc_body,
        out_type=jax.ShapeDtypeStruct((B,), jnp.float32),
        mesh=mesh,
        scratch_types=[
            pltpu.VMEM((F, BW), jnp.int32),
            pltpu.VMEM((F, BW), jnp.float32),
            pltpu.VMEM((BW,), jnp.float32),
            pltpu.VMEM((F, L), jnp.int32),
            pltpu.VMEM((L,), jnp.float32),
            pltpu.SemaphoreType.DMA,
        ],
    )(xt, fcr, offb, biasb)


def kernel(x, fc, bias, offsets):
    xt = x.T.reshape(F, NW, BW).transpose(1, 0, 2)
    offb = jnp.broadcast_to(offsets[:, None], (F, L))
    biasb = jnp.broadcast_to(bias, (L,))
    out = _features_linear(xt, fc.reshape(1, -1), offb, biasb)
    return out.reshape(B, 1)


# speedup vs baseline: 4.1655x; 1.0384x over previous
import functools

import jax
import jax.numpy as jnp
from jax import lax
from jax.experimental import pallas as pl
from jax.experimental.pallas import tpu as pltpu
from jax.experimental.pallas import tpu_sc as plsc

B = 16384
F = 26
NC, NS, L = 2, 16, 16
NW = NC * NS
BW = B // NW
NIDX = F * BW
CHUNK = 128
NCH = BW // CHUNK


def _sc_body(xt_hbm, fc_hbm, offb_hbm, biasb_hbm, out_hbm,
             xv, rows, outv, offv, biasv, sem):
    wid = lax.axis_index("s") * NC + lax.axis_index("c")
    base = wid * BW

    pltpu.sync_copy(xt_hbm.at[wid], xv)
    pltpu.sync_copy(offb_hbm, offv)
    pltpu.sync_copy(biasb_hbm, biasv)

    for f in range(F):
        off_f = offv[f, :]
        for c4 in range(NCH):
            @pl.loop(c4 * (CHUNK // L), (c4 + 1) * (CHUNK // L))
            def _idx(c):
                xv[f, pl.ds(c * L, L)] = xv[f, pl.ds(c * L, L)] + off_f
            pltpu.make_async_copy(
                fc_hbm.at[xv.at[pl.ds(f, 1), pl.ds(c4 * CHUNK, CHUNK)]],
                rows.at[pl.ds(f, 1), pl.ds(c4 * CHUNK, CHUNK)],
                sem,
            ).start()

    @pl.loop(0, F)
    def _drain(f):
        for c4 in range(NCH):
            pltpu.make_async_copy(
                fc_hbm.at[xv.at[pl.ds(f, 1), pl.ds(c4 * CHUNK, CHUNK)]],
                rows.at[pl.ds(f, 1), pl.ds(c4 * CHUNK, CHUNK)],
                sem,
            ).wait()

    bias_vec = biasv[:]
    @pl.loop(0, BW // L)
    def _reduce(c):
        acc = bias_vec
        for f in range(F):
            acc = acc + rows[f, pl.ds(c * L, L)]
        outv[pl.ds(c * L, L)] = acc

    pltpu.sync_copy(outv, out_hbm.at[pl.ds(base, BW)])


@jax.jit
def _features_linear(xt, fcr, offb, biasb):
    mesh = plsc.VectorSubcoreMesh(core_axis_name="c", subcore_axis_name="s")
    return pl.kernel(
        _sc_body,
        out_type=jax.ShapeDtypeStruct((B,), jnp.float32),
        mesh=mesh,
        scratch_types=[
            pltpu.VMEM((F, BW), jnp.int32),
            pltpu.VMEM((F, BW), jnp.float32),
            pltpu.VMEM((BW,), jnp.float32),
            pltpu.VMEM((F, L), jnp.int32),
            pltpu.VMEM((L,), jnp.float32),
            pltpu.SemaphoreType.DMA,
        ],
    )(xt, fcr, offb, biasb)


def kernel(x, fc, bias, offsets):
    xt = x.T.reshape(F, NW, BW).transpose(1, 0, 2)
    offb = jnp.broadcast_to(offsets[:, None], (F, L))
    biasb = jnp.broadcast_to(bias, (L,))
    out = _features_linear(xt, fc.reshape(1, -1), offb, biasb)
    return out.reshape(B, 1)
